# baseline (speedup 1.0000x reference)
import jax
import jax.numpy as jnp
from jax import lax
from jax.experimental import pallas as pl
from jax.experimental.pallas import tpu as pltpu
from jax.experimental.pallas import tpu_sc as plsc

N = 10000
E = 320000
D = 128

NC = 2
NS = 16
LANES = 16

CP = 128
S = 16
NSUPER = 10
CPT = NSUPER * S
EROWS = NS * CPT
NFAKE = EROWS * CP - E
TR8 = CP // LANES

RC = 128
NFULL = N // RC
TAIL = N - NFULL * RC
MAXK = (NFULL + NS - 1) // NS
NDEG = NFULL + 1
ND = NDEG * RC
NACC = ND


def _make_sc_kernel():
  mesh = plsc.VectorSubcoreMesh(core_axis_name="c", subcore_axis_name="s",
                                num_cores=NC, num_subcores=NS)

  def body(x_hbm, perm_hbm, src0, dst0, src1, dst1,
           sums_out, deg_out, xp_out,
           acc_sp, deg_sp,
           pidx_v, src_sb, dst_sb, rows_v, ones_v, degbuf_v,
           gsem0, gsem1, ssem0, ssem1, isem, sem):
    s = lax.axis_index("s")
    c = lax.axis_index("c")

    for q in range(TR8):
      ones_v[pl.ds(q * LANES, LANES)] = jnp.ones((LANES,), jnp.float32)
      degbuf_v[pl.ds(q * LANES, LANES)] = jnp.zeros((LANES,), jnp.float32)

    def _zero_rows(t, _):
      rows_v[0, t // 8, pl.ds((t % 8) * LANES, LANES)] = (
          jnp.zeros((LANES,), jnp.float32))
      return 0

    def _for_my_chunks(fn):
      def step(i, _):
        k = s + i * NS
        @pl.when(k < NFULL)
        def _():
          fn(k * RC, RC)
        return 0
      lax.fori_loop(0, MAXK, step, 0)
      @pl.when(s == 0)
      def _():
        fn(NFULL * RC, TAIL)

    def _for_deg_chunks(fn):
      def step(i, _):
        k = s + i * NS
        @pl.when(k < NDEG)
        def _():
          fn(k * RC)
        return 0
      lax.fori_loop(0, MAXK, step, 0)

    def _xp_chunk(off, sz):
      pltpu.sync_copy(perm_hbm.at[pl.ds(off, sz)], pidx_v.at[pl.ds(0, sz)])
      idx = pidx_v.at[pl.ds(0, sz)]
      pltpu.async_copy(x_hbm.at[idx], rows_v.at[0, pl.ds(0, sz)], sem).wait()
      pltpu.sync_copy(rows_v.at[0, pl.ds(0, sz)], xp_out.at[pl.ds(off, sz)])

    def _run_core(e, src2d, dst2d):
      _for_my_chunks(_xp_chunk)

      trow = s * CPT

      for v in range(2):
        lax.fori_loop(0, RC * (D // LANES), _zero_rows, 0)

        def _zero_chunk(off, sz):
          pltpu.sync_copy(rows_v.at[0, pl.ds(0, sz)], acc_sp.at[pl.ds(off, sz)])
        _for_my_chunks(_zero_chunk)
        if v == 0:
          _for_deg_chunks(
              lambda off: pltpu.sync_copy(degbuf_v, deg_sp.at[pl.ds(off, RC)]))
        plsc.subcore_barrier()

        tab = x_hbm if v == 0 else xp_out

        def _prep(sb, q, b, gsem):
          pltpu.async_copy(tab.at[src_sb.at[sb, q]], rows_v.at[b], gsem)

        def _wait_gather(sb, q, b, gsem):
          pltpu.make_async_copy(
              tab.at[src_sb.at[sb, q]], rows_v.at[b], gsem).wait()

        def _fire_scatter(sb, q, b, ssem):
          pltpu.async_copy(rows_v.at[b], acc_sp.at[dst_sb.at[sb, q]], ssem,
                           add=True)
          if v == 0:
            pltpu.async_copy(ones_v, deg_sp.at[dst_sb.at[sb, q]], ssem,
                             add=True)

        def _drain_scatter(sb, q, b, ssem):
          pltpu.make_async_copy(
              rows_v.at[b], acc_sp.at[dst_sb.at[sb, q]], ssem).wait()
          if v == 0:
            pltpu.make_async_copy(
                ones_v, deg_sp.at[dst_sb.at[sb, q]], ssem).wait()

        pltpu.sync_copy(src2d.at[pl.ds(trow, S)], src_sb.at[0])
        pltpu.sync_copy(dst2d.at[pl.ds(trow, S)], dst_sb.at[0])
        _prep(0, 0, 0, gsem0)

        def _super_pair(s4, _):
          for sbp in (0, 1):
            p, op = sbp, 1 - sbp
            sup = 2 * s4 + sbp
            r0 = trow + S * sup
            for qq in range(S // 2):
              cA, cB = 2 * qq, 2 * qq + 1
              if qq == 0:
                if sbp == 0:
                  @pl.when(s4 > 0)
                  def _():
                    _drain_scatter(op, S - 1, 1, ssem1)
                else:
                  _drain_scatter(op, S - 1, 1, ssem1)
                @pl.when(sup < NSUPER - 1)
                def _():
                  pltpu.async_copy(src2d.at[pl.ds(r0 + S, S)],
                                   src_sb.at[op], isem)
                  pltpu.async_copy(dst2d.at[pl.ds(r0 + S, S)],
                                   dst_sb.at[op], isem)
              else:
                _drain_scatter(p, cA - 1, 1, ssem1)
              _prep(p, cB, 1, gsem1)
              _wait_gather(p, cA, 0, gsem0)
              _fire_scatter(p, cA, 0, ssem0)
              _wait_gather(p, cB, 1, gsem1)
              _fire_scatter(p, cB, 1, ssem1)
              _drain_scatter(p, cA, 0, ssem0)
              if qq < S // 2 - 1:
                _prep(p, cA + 2, 0, gsem0)
              else:
                @pl.when(sup < NSUPER - 1)
                def _():
                  pltpu.make_async_copy(src2d.at[pl.ds(r0 + S, S)],
                                        src_sb.at[op], isem).wait()
                  pltpu.make_async_copy(dst2d.at[pl.ds(r0 + S, S)],
                                        dst_sb.at[op], isem).wait()
                  _prep(op, 0, 0, gsem0)
          return 0
        lax.fori_loop(0, NSUPER // 2, _super_pair, 0)
        _drain_scatter(1, S - 1, 1, ssem1)
        plsc.subcore_barrier()

        slot = v * 2 + e

        def _flush_chunk(off, sz):
          pltpu.sync_copy(acc_sp.at[pl.ds(off, sz)],
                          sums_out.at[slot, pl.ds(off, sz)])
        _for_my_chunks(_flush_chunk)
        if v == 0:
          def _flush_deg(off):
            pltpu.sync_copy(deg_sp.at[pl.ds(off, RC)],
                            deg_out.at[e, pl.ds(off, RC)])
          _for_deg_chunks(_flush_deg)
        plsc.subcore_barrier()

    @pl.when(c == 0)
    def _():
      _run_core(0, src0, dst0)

    @pl.when(c == 1)
    def _():
      _run_core(1, src1, dst1)

  return pl.kernel(
      body,
      out_type=[
          jax.ShapeDtypeStruct((4, N, D), jnp.float32),
          jax.ShapeDtypeStruct((NC, ND), jnp.float32),
          jax.ShapeDtypeStruct((N, D), jnp.float32),
      ],
      mesh=mesh,
      compiler_params=pltpu.CompilerParams(needs_layout_passes=False),
      scratch_types=[
          pltpu.VMEM_SHARED((NACC, D), jnp.float32),
          pltpu.VMEM_SHARED((ND,), jnp.float32),
          pltpu.VMEM((RC,), jnp.int32),
          pltpu.VMEM((2, S, CP), jnp.int32),
          pltpu.VMEM((2, S, CP), jnp.int32),
          pltpu.VMEM((2, CP, D), jnp.float32),
          pltpu.VMEM((CP,), jnp.float32),
          pltpu.VMEM((RC,), jnp.float32),
          pltpu.SemaphoreType.DMA,
          pltpu.SemaphoreType.DMA,
          pltpu.SemaphoreType.DMA,
          pltpu.SemaphoreType.DMA,
          pltpu.SemaphoreType.DMA,
          pltpu.SemaphoreType.DMA,
      ],
  )


_sc_kernel = _make_sc_kernel()

BR = 1000


def _tc_body(sums_ref, deg_ref, x_ref, xp_ref, wl_ref, wr_ref, b_ref, out_ref):
  x = x_ref[...]
  xp = xp_ref[...]
  for e in range(2):
    wl = wl_ref[e]
    wr = wr_ref[e]
    bb = b_ref[e]
    dinv = 1.0 / jnp.maximum(deg_ref[e], 1.0)
    root_pos = jnp.dot(x, wr, preferred_element_type=jnp.float32)
    root_neg = jnp.dot(xp, wr, preferred_element_type=jnp.float32)
    agg_pos = sums_ref[e] * dinv
    agg_neg = sums_ref[2 + e] * dinv
    out_ref[e] = jnp.maximum(
        jnp.dot(agg_pos, wl, preferred_element_type=jnp.float32) + bb + root_pos,
        0.0)
    out_ref[2 + e] = jnp.maximum(
        jnp.dot(agg_neg, wl, preferred_element_type=jnp.float32) + bb + root_neg,
        0.0)


def _tc_kernel(sums, deg3, x, xp, wlT, wrT, b3):
  return pl.pallas_call(
      _tc_body,
      grid=(N // BR,),
      in_specs=[
          pl.BlockSpec((4, BR, D), lambda i: (0, i, 0)),
          pl.BlockSpec((2, BR, 1), lambda i: (0, i, 0)),
          pl.BlockSpec((BR, D), lambda i: (i, 0)),
          pl.BlockSpec((BR, D), lambda i: (i, 0)),
          pl.BlockSpec((2, D, D), lambda i: (0, 0, 0)),
          pl.BlockSpec((2, D, D), lambda i: (0, 0, 0)),
          pl.BlockSpec((2, 1, D), lambda i: (0, 0, 0)),
      ],
      out_specs=pl.BlockSpec((4, BR, D), lambda i: (0, i, 0)),
      out_shape=jax.ShapeDtypeStruct((4, N, D), jnp.float32),
  )(sums, deg3, x, xp, wlT, wrT, b3)


def _pad_edges(ei):
  fake = jnp.arange(NFAKE, dtype=jnp.int32)
  src = jnp.concatenate([ei[0], fake % N]).reshape(EROWS, CP)
  dst = jnp.concatenate([ei[1], N + fake % (NACC - N)]).reshape(EROWS, CP)
  return src, dst


@jax.jit
def kernel(x, W_l0, b_l0, W_r0, W_l1, b_l1, W_r1,
           edge_index_0, edge_index_1, perm):
  src0, dst0 = _pad_edges(edge_index_0)
  src1, dst1 = _pad_edges(edge_index_1)
  perm = perm.astype(jnp.int32)

  sums, deg, xp = _sc_kernel(x, perm, src0, dst0, src1, dst1)

  wlT = jnp.stack([W_l0.T, W_l1.T])
  wrT = jnp.stack([W_r0.T, W_r1.T])
  b3 = jnp.stack([b_l0, b_l1])[:, None, :]
  deg3 = deg[:, :N, None]

  return _tc_kernel(sums, deg3, x, xp, wlT, wrT, b3)


# speedup vs baseline: 1.1165x; 1.1165x over previous
import jax
import jax.numpy as jnp
from jax import lax
from jax.experimental import pallas as pl
from jax.experimental.pallas import tpu as pltpu
from jax.experimental.pallas import tpu_sc as plsc

N = 10000
E = 320000
D = 128

NC = 2
NS = 16
LANES = 16

CP = 128
S = 16
NSUPER = 10
CPT = NSUPER * S
EROWS = NS * CPT
NFAKE = EROWS * CP - E
TR8 = CP // LANES

RC = 128
NFULL = N // RC
TAIL = N - NFULL * RC
MAXK = (NFULL + NS - 1) // NS
NDEG = NFULL + 1
ND = NDEG * RC
NACC = ND


def _make_sc_kernel():
  mesh = plsc.VectorSubcoreMesh(core_axis_name="c", subcore_axis_name="s",
                                num_cores=NC, num_subcores=NS)

  def body(x_hbm, perm_hbm, src0, dst0, src1, dst1,
           sums_out, deg_out, xp_out,
           acc_sp, deg_sp,
           pidx_v, src_sb, dst_sb, rows_v, ones_v, degbuf_v,
           gsem0, gsem1, ssem0, ssem1, isem, sem):
    s = lax.axis_index("s")
    c = lax.axis_index("c")

    for q in range(TR8):
      ones_v[pl.ds(q * LANES, LANES)] = jnp.ones((LANES,), jnp.float32)
      degbuf_v[pl.ds(q * LANES, LANES)] = jnp.zeros((LANES,), jnp.float32)

    def _zero_rows(t, _):
      rows_v[0, t // 8, pl.ds((t % 8) * LANES, LANES)] = (
          jnp.zeros((LANES,), jnp.float32))
      return 0

    def _for_my_chunks(fn):
      def step(i, _):
        k = s + i * NS
        @pl.when(k < NFULL)
        def _():
          fn(k * RC, RC)
        return 0
      lax.fori_loop(0, MAXK, step, 0)
      @pl.when(s == 0)
      def _():
        fn(NFULL * RC, TAIL)

    def _for_deg_chunks(fn):
      def step(i, _):
        k = s + i * NS
        @pl.when(k < NDEG)
        def _():
          fn(k * RC)
        return 0
      lax.fori_loop(0, MAXK, step, 0)

    def _xp_chunk(off, sz):
      pltpu.sync_copy(perm_hbm.at[pl.ds(off, sz)], pidx_v.at[pl.ds(0, sz)])
      idx = pidx_v.at[pl.ds(0, sz)]
      pltpu.async_copy(x_hbm.at[idx], rows_v.at[0, pl.ds(0, sz)], sem).wait()
      pltpu.sync_copy(rows_v.at[0, pl.ds(0, sz)], xp_out.at[pl.ds(off, sz)])

    def _run_core(e, src2d, dst2d):
      _for_my_chunks(_xp_chunk)

      trow = s * CPT

      for v in range(2):
        lax.fori_loop(0, RC * (D // LANES), _zero_rows, 0)

        def _zero_chunk(off, sz):
          pltpu.sync_copy(rows_v.at[0, pl.ds(0, sz)], acc_sp.at[pl.ds(off, sz)])
        _for_my_chunks(_zero_chunk)
        if v == 0:
          _for_deg_chunks(
              lambda off: pltpu.sync_copy(degbuf_v, deg_sp.at[pl.ds(off, RC)]))
        plsc.subcore_barrier()

        tab = x_hbm if v == 0 else xp_out

        def _prep(sb, q, b, gsem):
          pltpu.async_copy(tab.at[src_sb.at[sb, q]], rows_v.at[b], gsem)

        def _wait_gather(sb, q, b, gsem):
          pltpu.make_async_copy(
              tab.at[src_sb.at[sb, q]], rows_v.at[b], gsem).wait()

        def _fire_scatter(sb, q, b, ssem):
          pltpu.async_copy(rows_v.at[b], acc_sp.at[dst_sb.at[sb, q]], ssem,
                           add=True)
          if v == 0:
            pltpu.async_copy(ones_v, deg_sp.at[dst_sb.at[sb, q]], ssem,
                             add=True)

        def _drain_scatter(sb, q, b, ssem):
          pltpu.make_async_copy(
              rows_v.at[b], acc_sp.at[dst_sb.at[sb, q]], ssem).wait()
          if v == 0:
            pltpu.make_async_copy(
                ones_v, deg_sp.at[dst_sb.at[sb, q]], ssem).wait()

        pltpu.sync_copy(src2d.at[pl.ds(trow, S)], src_sb.at[0])
        pltpu.sync_copy(dst2d.at[pl.ds(trow, S)], dst_sb.at[0])
        _prep(0, 0, 0, gsem0)

        def _super_pair(s4, _):
          for sbp in (0, 1):
            p, op = sbp, 1 - sbp
            sup = 2 * s4 + sbp
            r0 = trow + S * sup
            for qq in range(S // 2):
              cA, cB = 2 * qq, 2 * qq + 1
              _wait_gather(p, cA, 0, gsem0)
              _fire_scatter(p, cA, 0, ssem0)
              if qq == 0:
                if sbp == 0:
                  @pl.when(s4 > 0)
                  def _():
                    _drain_scatter(op, S - 1, 1, ssem1)
                else:
                  _drain_scatter(op, S - 1, 1, ssem1)
                @pl.when(sup < NSUPER - 1)
                def _():
                  pltpu.async_copy(src2d.at[pl.ds(r0 + S, S)],
                                   src_sb.at[op], isem)
                  pltpu.async_copy(dst2d.at[pl.ds(r0 + S, S)],
                                   dst_sb.at[op], isem)
              else:
                _drain_scatter(p, cA - 1, 1, ssem1)
              _prep(p, cB, 1, gsem1)
              _wait_gather(p, cB, 1, gsem1)
              _fire_scatter(p, cB, 1, ssem1)
              _drain_scatter(p, cA, 0, ssem0)
              if qq < S // 2 - 1:
                _prep(p, cA + 2, 0, gsem0)
              else:
                @pl.when(sup < NSUPER - 1)
                def _():
                  pltpu.make_async_copy(src2d.at[pl.ds(r0 + S, S)],
                                        src_sb.at[op], isem).wait()
                  pltpu.make_async_copy(dst2d.at[pl.ds(r0 + S, S)],
                                        dst_sb.at[op], isem).wait()
                  _prep(op, 0, 0, gsem0)
          return 0
        lax.fori_loop(0, NSUPER // 2, _super_pair, 0)
        _drain_scatter(1, S - 1, 1, ssem1)
        plsc.subcore_barrier()

        slot = v * 2 + e

        def _flush_chunk(off, sz):
          pltpu.sync_copy(acc_sp.at[pl.ds(off, sz)],
                          sums_out.at[slot, pl.ds(off, sz)])
        _for_my_chunks(_flush_chunk)
        if v == 0:
          def _flush_deg(off):
            pltpu.sync_copy(deg_sp.at[pl.ds(off, RC)],
                            deg_out.at[e, pl.ds(off, RC)])
          _for_deg_chunks(_flush_deg)
        plsc.subcore_barrier()

    @pl.when(c == 0)
    def _():
      _run_core(0, src0, dst0)

    @pl.when(c == 1)
    def _():
      _run_core(1, src1, dst1)

  return pl.kernel(
      body,
      out_type=[
          jax.ShapeDtypeStruct((4, N, D), jnp.float32),
          jax.ShapeDtypeStruct((NC, ND), jnp.float32),
          jax.ShapeDtypeStruct((N, D), jnp.float32),
      ],
      mesh=mesh,
      compiler_params=pltpu.CompilerParams(needs_layout_passes=False),
      scratch_types=[
          pltpu.VMEM_SHARED((NACC, D), jnp.float32),
          pltpu.VMEM_SHARED((ND,), jnp.float32),
          pltpu.VMEM((RC,), jnp.int32),
          pltpu.VMEM((2, S, CP), jnp.int32),
          pltpu.VMEM((2, S, CP), jnp.int32),
          pltpu.VMEM((2, CP, D), jnp.float32),
          pltpu.VMEM((CP,), jnp.float32),
          pltpu.VMEM((RC,), jnp.float32),
          pltpu.SemaphoreType.DMA,
          pltpu.SemaphoreType.DMA,
          pltpu.SemaphoreType.DMA,
          pltpu.SemaphoreType.DMA,
          pltpu.SemaphoreType.DMA,
          pltpu.SemaphoreType.DMA,
      ],
  )


_sc_kernel = _make_sc_kernel()

BR = 1000


def _tc_body(sums_ref, deg_ref, x_ref, xp_ref, wl_ref, wr_ref, b_ref, out_ref):
  x = x_ref[...]
  xp = xp_ref[...]
  for e in range(2):
    wl = wl_ref[e]
    wr = wr_ref[e]
    bb = b_ref[e]
    dinv = 1.0 / jnp.maximum(deg_ref[e], 1.0)
    root_pos = jnp.dot(x, wr, preferred_element_type=jnp.float32)
    root_neg = jnp.dot(xp, wr, preferred_element_type=jnp.float32)
    agg_pos = sums_ref[e] * dinv
    agg_neg = sums_ref[2 + e] * dinv
    out_ref[e] = jnp.maximum(
        jnp.dot(agg_pos, wl, preferred_element_type=jnp.float32) + bb + root_pos,
        0.0)
    out_ref[2 + e] = jnp.maximum(
        jnp.dot(agg_neg, wl, preferred_element_type=jnp.float32) + bb + root_neg,
        0.0)


def _tc_kernel(sums, deg3, x, xp, wlT, wrT, b3):
  return pl.pallas_call(
      _tc_body,
      grid=(N // BR,),
      in_specs=[
          pl.BlockSpec((4, BR, D), lambda i: (0, i, 0)),
          pl.BlockSpec((2, BR, 1), lambda i: (0, i, 0)),
          pl.BlockSpec((BR, D), lambda i: (i, 0)),
          pl.BlockSpec((BR, D), lambda i: (i, 0)),
          pl.BlockSpec((2, D, D), lambda i: (0, 0, 0)),
          pl.BlockSpec((2, D, D), lambda i: (0, 0, 0)),
          pl.BlockSpec((2, 1, D), lambda i: (0, 0, 0)),
      ],
      out_specs=pl.BlockSpec((4, BR, D), lambda i: (0, i, 0)),
      out_shape=jax.ShapeDtypeStruct((4, N, D), jnp.float32),
  )(sums, deg3, x, xp, wlT, wrT, b3)


def _pad_edges(ei):
  fake = jnp.arange(NFAKE, dtype=jnp.int32)
  src = jnp.concatenate([ei[0], fake % N]).reshape(EROWS, CP)
  dst = jnp.concatenate([ei[1], N + fake % (NACC - N)]).reshape(EROWS, CP)
  return src, dst


@jax.jit
def kernel(x, W_l0, b_l0, W_r0, W_l1, b_l1, W_r1,
           edge_index_0, edge_index_1, perm):
  src0, dst0 = _pad_edges(edge_index_0)
  src1, dst1 = _pad_edges(edge_index_1)
  perm = perm.astype(jnp.int32)

  sums, deg, xp = _sc_kernel(x, perm, src0, dst0, src1, dst1)

  wlT = jnp.stack([W_l0.T, W_l1.T])
  wrT = jnp.stack([W_r0.T, W_r1.T])
  b3 = jnp.stack([b_l0, b_l1])[:, None, :]
  deg3 = deg[:, :N, None]

  return _tc_kernel(sums, deg3, x, xp, wlT, wrT, b3)
